# baseline (speedup 1.0000x reference)
import functools

import jax
import jax.numpy as jnp
from jax import lax
from jax.experimental import pallas as pl
from jax.experimental.pallas import tpu as pltpu
from jax.experimental.pallas import tpu_sc as plsc

H = 2
C = 32
NC = 2
NS = 16
BLK = 1568
KA = 2048
KB = 256
NEG = -1e30


def _sc_mesh():
    return plsc.VectorSubcoreMesh(
        core_axis_name="c", subcore_axis_name="s", num_cores=NC, num_subcores=NS)


_SC_PARAMS = pltpu.CompilerParams(
    needs_layout_passes=False, use_tc_tiling_on_sc=False)


def _stats(hflat, nvalid):
    def body(x_ref, o_ref):
        x = x_ref[...]
        r = lax.broadcasted_iota(jnp.int32, x.shape, 0)
        l = lax.broadcasted_iota(jnp.int32, x.shape, 1)
        g = r * 128 + l
        valid = g < nvalid
        even = (g % 2) == 0
        x0 = jnp.where(valid & even, x, 0.0)
        x1 = jnp.where(valid & (~even), x, 0.0)
        s0, s1 = jnp.sum(x0), jnp.sum(x1)
        q0, q1 = jnp.sum(x0 * x0), jnp.sum(x1 * x1)
        lane = lax.broadcasted_iota(jnp.int32, (1, 128), 1)
        o_ref[...] = jnp.where(
            lane == 0, s0, jnp.where(lane == 1, s1, jnp.where(lane == 2, q0, q1)))
    return pl.pallas_call(
        body, out_shape=jax.ShapeDtypeStruct((1, 128), jnp.float32))(hflat)


def _proj1(hpad, P, rrow, npad):
    def body(x_ref, p_ref, r_ref, xw0_ref, xw1_ref, th_ref, m_ref):
        y = jnp.dot(x_ref[...], p_ref[...],
                    preferred_element_type=jnp.float32, precision=lax.Precision.HIGHEST) + r_ref[...]
        xw0_ref[...] = y[:, 0:32]
        xw1_ref[...] = y[:, 32:64]
        th_ref[...] = y[:, 64:68]
        cur = jnp.max(y, axis=0, keepdims=True)
        @pl.when(pl.program_id(0) == 0)
        def _():
            m_ref[...] = cur
        @pl.when(pl.program_id(0) != 0)
        def _():
            m_ref[...] = jnp.maximum(m_ref[...], cur)
    grid = (npad // BLK,)
    return pl.pallas_call(
        body, grid=grid,
        in_specs=[
            pl.BlockSpec((BLK, 2), lambda i: (i, 0)),
            pl.BlockSpec((2, 128), lambda i: (0, 0)),
            pl.BlockSpec((1, 128), lambda i: (0, 0)),
        ],
        out_specs=[
            pl.BlockSpec((BLK, 32), lambda i: (i, 0)),
            pl.BlockSpec((BLK, 32), lambda i: (i, 0)),
            pl.BlockSpec((BLK, 4), lambda i: (i, 0)),
            pl.BlockSpec((1, 128), lambda i: (0, 0)),
        ],
        out_shape=[
            jax.ShapeDtypeStruct((npad, 32), jnp.float32),
            jax.ShapeDtypeStruct((npad, 32), jnp.float32),
            jax.ShapeDtypeStruct((npad, 4), jnp.float32),
            jax.ShapeDtypeStruct((1, 128), jnp.float32),
        ])(hpad, P, rrow)


def _proj2(agg, brow, P, npad):
    nb = npad // BLK
    def body(a0_ref, a1_ref, b_ref, p_ref, xw0_ref, xw1_ref, th_ref, m_ref):
        x = jnp.concatenate([a0_ref[...], a1_ref[...]], axis=1) + b_ref[...]
        x = jnp.maximum(x, 0.0)
        y = jnp.dot(x, p_ref[...], preferred_element_type=jnp.float32, precision=lax.Precision.HIGHEST)
        xw0_ref[...] = y[:, 0:32]
        xw1_ref[...] = y[:, 32:64]
        th_ref[...] = y[:, 64:68]
        cur = jnp.max(y, axis=0, keepdims=True)
        @pl.when(pl.program_id(0) == 0)
        def _():
            m_ref[...] = cur
        @pl.when(pl.program_id(0) != 0)
        def _():
            m_ref[...] = jnp.maximum(m_ref[...], cur)
    return pl.pallas_call(
        body, grid=(nb,),
        in_specs=[
            pl.BlockSpec((BLK, 32), lambda i: (i, 0)),
            pl.BlockSpec((BLK, 32), lambda i: (i + nb, 0)),
            pl.BlockSpec((1, 64), lambda i: (0, 0)),
            pl.BlockSpec((64, 128), lambda i: (0, 0)),
        ],
        out_specs=[
            pl.BlockSpec((BLK, 32), lambda i: (i, 0)),
            pl.BlockSpec((BLK, 32), lambda i: (i, 0)),
            pl.BlockSpec((BLK, 4), lambda i: (i, 0)),
            pl.BlockSpec((1, 128), lambda i: (0, 0)),
        ],
        out_shape=[
            jax.ShapeDtypeStruct((npad, 32), jnp.float32),
            jax.ShapeDtypeStruct((npad, 32), jnp.float32),
            jax.ShapeDtypeStruct((npad, 4), jnp.float32),
            jax.ShapeDtypeStruct((1, 128), jnp.float32),
        ])(agg, agg, brow, P)


def _mlp(agg, brow, Wf1, bf1, Wf2, bf2, Wf3, bf3, Wf4, bf4, Wf5, bf5, npad):
    nb = npad // BLK
    def body(a0_ref, a1_ref, b_ref, w1, c1, w2, c2, w3, c3, w4, c4, w5, c5,
             out_ref):
        x = jnp.concatenate([a0_ref[...], a1_ref[...]], axis=1) + b_ref[...]
        x = jnp.maximum(x, 0.0)
        for w, c in ((w1, c1), (w2, c2), (w3, c3), (w4, c4)):
            x = jnp.maximum(
                jnp.dot(x, w[...], preferred_element_type=jnp.float32, precision=lax.Precision.HIGHEST) + c[...],
                0.0)
        out_ref[...] = jnp.dot(
            x, w5[...], preferred_element_type=jnp.float32, precision=lax.Precision.HIGHEST) + c5[...]
    full = lambda s: pl.BlockSpec(s, lambda i: (0, 0))
    return pl.pallas_call(
        body, grid=(nb,),
        in_specs=[
            pl.BlockSpec((BLK, 32), lambda i: (i, 0)),
            pl.BlockSpec((BLK, 32), lambda i: (i + nb, 0)),
            full((1, 64)),
            full((64, 64)), full((1, 64)),
            full((64, 64)), full((1, 64)),
            full((64, 64)), full((1, 64)),
            full((64, 64)), full((1, 64)),
            full((64, 11)), full((1, 11)),
        ],
        out_specs=pl.BlockSpec((BLK, 11), lambda i: (i, 0)),
        out_shape=jax.ShapeDtypeStruct((npad, 11), jnp.float32),
    )(agg, agg, brow, Wf1, bf1.reshape(1, 64), Wf2, bf2.reshape(1, 64),
      Wf3, bf3.reshape(1, 64), Wf4, bf4.reshape(1, 64), Wf5, bf5.reshape(1, 11))


def _sc_logits(sa2, sd2, src1d, dst2d, mvecs, zn, npad, epad):
    nslice = npad // NS
    ept = epad // NS
    nchunks = ept // KA

    @functools.partial(
        pl.kernel,
        out_type=jax.ShapeDtypeStruct((NC * epad,), jnp.float32),
        mesh=_sc_mesh(),
        compiler_params=_SC_PARAMS,
        scratch_types=[
            pltpu.VMEM((npad,), jnp.float32),
            pltpu.VMEM((npad,), jnp.float32),
            pltpu.VMEM((KA,), jnp.int32),
            pltpu.VMEM((KA,), jnp.int32),
            pltpu.VMEM((KA // 128, 128), jnp.int32),
            pltpu.VMEM((KA // 128, 128), jnp.int32),
            pltpu.VMEM((KA,), jnp.float32),
            pltpu.VMEM((KA,), jnp.float32),
            pltpu.VMEM((16,), jnp.float32),
            pltpu.VMEM((nslice,), jnp.float32),
            pltpu.VMEM_SHARED((npad,), jnp.float32),
            pltpu.SemaphoreType.DMA,
            pltpu.SemaphoreType.DMA,
        ])
    def kern(sa_h, sd_h, src_h, dst2_h, mv_h, zn_h, t_out,
             satab, sdtab, src0, src1, dst20, dst21, tb0, tb1, m_b,
             zslice, s_acc, ss0, ss1):
        c = lax.axis_index("c")
        s = lax.axis_index("s")
        pltpu.sync_copy(sa_h.at[pl.ds(c * npad, npad)], satab)
        pltpu.sync_copy(sd_h.at[pl.ds(c * npad, npad)], sdtab)
        pltpu.sync_copy(mv_h.at[pl.ds(c * 16, 16)], m_b)
        pltpu.sync_copy(zn_h, zslice)
        pltpu.sync_copy(zslice, s_acc.at[pl.ds(s * nslice, nslice)])
        plsc.subcore_barrier()
        mv = m_b[...]

        def load(k, src_b, dst2_b):
            off = pl.multiple_of(s * ept + k * KA, KA)
            row = s * (ept // 128) + k * (KA // 128)
            pltpu.sync_copy(src_h.at[pl.ds(off, KA)], src_b)
            pltpu.sync_copy(dst2_h.at[pl.ds(row, KA // 128), :], dst2_b)

        def compute(k, src_b, dst2_b, t_b, sem):
            off = pl.multiple_of(s * ept + k * KA, KA)

            def vec(v, carry2):
                o = pl.multiple_of(v * 16, 16)
                r = v // 8
                col = pl.multiple_of((v % 8) * 16, 16)
                a = plsc.load_gather(satab, [src_b[pl.ds(o, 16)]])
                d = plsc.load_gather(sdtab, [dst2_b[r, pl.ds(col, 16)]])
                e = a + d
                e = jnp.where(e > 0, e, 0.2 * e)
                t_b[pl.ds(o, 16)] = jnp.exp(e - mv)
                return carry2
            lax.fori_loop(0, KA // 16, vec, None)
            pltpu.sync_copy(t_b, t_out.at[pl.ds(c * epad + off, KA)])
            for b in range(KA // 128):
                pltpu.async_copy(t_b.at[pl.ds(b * 128, 128)],
                                 s_acc.at[dst2_b.at[b]], sem, add=True)

        def drain(t_b, sem):
            pltpu.make_async_copy(zn_h.at[pl.ds(0, KA)], t_b, sem).wait()

        load(0, src0, dst20)
        nch2 = nchunks // 2

        def body2(k2, carry):
            @pl.when(k2 > 0)
            def _():
                drain(tb1, ss1)
            load(2 * k2 + 1, src1, dst21)
            compute(2 * k2, src0, dst20, tb0, ss0)

            @pl.when(k2 + 1 < nch2)
            def _():
                drain(tb0, ss0)
                load(2 * k2 + 2, src0, dst20)
            compute(2 * k2 + 1, src1, dst21, tb1, ss1)
            return carry
        lax.fori_loop(0, nch2, body2, None)
        drain(tb0, ss0)
        drain(tb1, ss1)
        plsc.subcore_barrier()
        pltpu.sync_copy(s_acc, satab)

        def norm(k, dst2_b, t_b):
            off = pl.multiple_of(s * ept + k * KA, KA)
            pltpu.sync_copy(t_out.at[pl.ds(c * epad + off, KA)], t_b)

            def vec2(v, carry2):
                o = pl.multiple_of(v * 16, 16)
                r = v // 8
                col = pl.multiple_of((v % 8) * 16, 16)
                sv = plsc.load_gather(satab, [dst2_b[r, pl.ds(col, 16)]])
                t_b[pl.ds(o, 16)] = t_b[pl.ds(o, 16)] / (sv + 1e-16)
                return carry2
            lax.fori_loop(0, KA // 16, vec2, None)
            pltpu.sync_copy(t_b, t_out.at[pl.ds(c * epad + off, KA)])

        def load2(k, dst2_b):
            row = s * (ept // 128) + k * (KA // 128)
            pltpu.sync_copy(dst2_h.at[pl.ds(row, KA // 128), :], dst2_b)

        load2(0, dst20)

        def body2b(k2, carry):
            load2(2 * k2 + 1, dst21)
            norm(2 * k2, dst20, tb0)

            @pl.when(k2 + 1 < nch2)
            def _():
                load2(2 * k2 + 2, dst20)
            norm(2 * k2 + 1, dst21, tb1)
            return carry
        lax.fori_loop(0, nch2, body2b, None)

    return kern(sa2, sd2, src1d, dst2d, mvecs, zn)


def _sc_aggregate(xwcat, alpha, sidx2d, dst2d, zrows, npad, epad):
    nslice = npad // NS
    ept = epad // NS
    nchunks = ept // KB

    @functools.partial(
        pl.kernel,
        out_type=jax.ShapeDtypeStruct((NC * npad, 32), jnp.float32),
        mesh=_sc_mesh(),
        compiler_params=_SC_PARAMS,
        scratch_types=[
            pltpu.VMEM((KB,), jnp.float32),
            pltpu.VMEM((KB,), jnp.float32),
            pltpu.VMEM((KB // 128, 128), jnp.int32),
            pltpu.VMEM((KB // 128, 128), jnp.int32),
            pltpu.VMEM((KB // 128, 128), jnp.int32),
            pltpu.VMEM((KB // 128, 128), jnp.int32),
            pltpu.VMEM((KB, 32), jnp.float32),
            pltpu.VMEM((KB, 32), jnp.float32),
            pltpu.VMEM_SHARED((npad, 32), jnp.float32),
            pltpu.SemaphoreType.DMA,
            pltpu.SemaphoreType.DMA,
            pltpu.SemaphoreType.DMA,
            pltpu.SemaphoreType.DMA,
        ])
    def kern(xw_h, al_h, sidx_h, dst2_h, z_h, agg_out,
             al0, al1, d20, d21, si0, si1, rows0, rows1, acc,
             sg0, sg1, ss0, ss1):
        c = lax.axis_index("c")
        s = lax.axis_index("s")
        pltpu.sync_copy(z_h, rows0)
        for p in range(nslice // 224):
            pltpu.sync_copy(rows0.at[pl.ds(0, 224), :],
                            acc.at[pl.ds(s * nslice + p * 224, 224), :])
        plsc.subcore_barrier()

        def load(k, al_b, dst2_b, sidx_b, rows, sem):
            off = pl.multiple_of(s * ept + k * KB, KB)
            row = s * (ept // 128) + k * (KB // 128)
            pltpu.sync_copy(al_h.at[pl.ds(c * epad + off, KB)], al_b)
            pltpu.sync_copy(dst2_h.at[pl.ds(row, KB // 128), :], dst2_b)
            pltpu.sync_copy(
                sidx_h.at[pl.ds(c * (epad // 128) + row, KB // 128), :], sidx_b)
            for b in range(KB // 128):
                pltpu.async_copy(xw_h.at[sidx_b.at[b]],
                                 rows.at[pl.ds(b * 128, 128), :], sem)

        def drain(rows, sem):
            pltpu.make_async_copy(z_h, rows, sem).wait()

        def scale_scatter(al_b, dst2_b, rows, sem):
            def scale_g(g, carry2):
                o = pl.multiple_of(g * 16, 16)
                al = al_b[pl.ds(o, 16)]
                for j in range(16):
                    e = g * 16 + j
                    sp = al.at[jnp.full((16,), j, jnp.int32)].get(
                        mode="promise_in_bounds")
                    rows[e, pl.ds(0, 16)] = rows[e, pl.ds(0, 16)] * sp
                    rows[e, pl.ds(16, 16)] = rows[e, pl.ds(16, 16)] * sp
                return carry2
            lax.fori_loop(0, KB // 16, scale_g, None)
            for b in range(KB // 128):
                pltpu.async_copy(rows.at[pl.ds(b * 128, 128), :],
                                 acc.at[dst2_b.at[b]], sem, add=True)

        load(0, al0, d20, si0, rows0, sg0)
        nch2 = nchunks // 2

        def body2(k2, carry):
            @pl.when(k2 > 0)
            def _():
                drain(rows1, ss1)
            load(2 * k2 + 1, al1, d21, si1, rows1, sg1)
            drain(rows0, sg0)
            scale_scatter(al0, d20, rows0, ss0)

            @pl.when(k2 + 1 < nch2)
            def _():
                drain(rows0, ss0)
                load(2 * k2 + 2, al0, d20, si0, rows0, sg0)
            drain(rows1, sg1)
            scale_scatter(al1, d21, rows1, ss1)
            return carry
        lax.fori_loop(0, nch2, body2, None)
        drain(rows0, ss0)
        drain(rows1, ss1)
        plsc.subcore_barrier()
        for p in range(nslice // 224):
            pltpu.sync_copy(acc.at[pl.ds(s * nslice + p * 224, 224), :],
                            rows0.at[pl.ds(0, 224), :])
            pltpu.sync_copy(
                rows0.at[pl.ds(0, 224), :],
                agg_out.at[pl.ds(c * npad + s * nslice + p * 224, 224), :])

    return kern(xwcat, alpha, sidx2d, dst2d, zrows)


def _leaky(v):
    return jnp.where(v > 0, v, 0.2 * v)


def _attn_cols(W, a_s, a_d):
    cols = []
    for h in range(H):
        cols.append(W[:, h * C:(h + 1) * C] @ a_s[h])
        cols.append(W[:, h * C:(h + 1) * C] @ a_d[h])
    return jnp.stack(cols, axis=1)


def _tables(th, m, valid, npad):
    sa2 = jnp.concatenate([jnp.where(valid, th[:, 0], NEG),
                           jnp.where(valid, th[:, 2], NEG)])
    sd2 = jnp.concatenate([jnp.where(valid, th[:, 1], 0.0),
                           jnp.where(valid, th[:, 3], 0.0)])
    m0 = _leaky(m[0, 64] + m[0, 65])
    m1 = _leaky(m[0, 66] + m[0, 67])
    mvecs = jnp.concatenate([jnp.full((16,), m0, jnp.float32),
                             jnp.full((16,), m1, jnp.float32)])
    return sa2, sd2, mvecs


def kernel(h, edge_index, bn_g, bn_b, W1, as1, ad1, b1, W2, as2, ad2, b2,
           Wf1, bf1, Wf2, bf2, Wf3, bf3, Wf4, bf4, Wf5, bf5):
    n = h.shape[0]
    npad = ((n + 2 * BLK - 1) // (2 * BLK)) * (2 * BLK)
    e_tot = edge_index.shape[1] + n
    epad = ((e_tot + NS * KA - 1) // (NS * KA)) * (NS * KA)

    loops = jnp.arange(n, dtype=jnp.int32)
    pad_e = jnp.full((epad - e_tot,), n, jnp.int32)
    src1d = jnp.concatenate([edge_index[0].astype(jnp.int32), loops, pad_e])
    dst1d = jnp.concatenate([edge_index[1].astype(jnp.int32), loops, pad_e])
    dst2d = dst1d.reshape(epad // 128, 128)
    sidx2d = jnp.concatenate([src1d, src1d + npad]).reshape(2 * epad // 128, 128)
    zn = jnp.zeros((npad // NS,), jnp.float32)
    zrows = jnp.zeros((KB, 32), jnp.float32)
    valid = jnp.arange(npad) < n

    hpad = jnp.pad(h, ((0, npad - n), (0, 0)))
    stats = _stats(hpad.reshape(npad * 2 // 128, 128), 2 * n)
    mean = jnp.stack([stats[0, 0], stats[0, 1]]) / n
    msq = jnp.stack([stats[0, 2], stats[0, 3]]) / n
    var = msq - mean * mean
    scale = bn_g / jnp.sqrt(var + 1e-5)
    shift = bn_b - mean * scale
    W1e = W1 * scale[:, None]
    P1 = jnp.concatenate([W1e, _attn_cols(W1e, as1, ad1)], axis=1)
    P1 = jnp.pad(P1, ((0, 0), (0, 128 - P1.shape[1])))
    r_xw = shift @ W1
    r_att = jnp.stack([r_xw[0:32] @ as1[0], r_xw[0:32] @ ad1[0],
                       r_xw[32:64] @ as1[1], r_xw[32:64] @ ad1[1]])
    rrow = jnp.pad(jnp.concatenate([r_xw, r_att]), (0, 60)).reshape(1, 128)

    xw0, xw1, th1, m1 = _proj1(hpad, P1, rrow, npad)
    sa2, sd2, mvecs = _tables(th1, m1, valid, npad)
    al1 = _sc_logits(sa2, sd2, src1d, dst2d, mvecs, zn, npad, epad)
    xwcat = jnp.concatenate([xw0, xw1], axis=0)
    agg1 = _sc_aggregate(xwcat, al1, sidx2d, dst2d, zrows, npad, epad)

    P2 = jnp.concatenate([W2, _attn_cols(W2, as2, ad2)], axis=1)
    P2 = jnp.pad(P2, ((0, 0), (0, 128 - P2.shape[1])))
    xw0b, xw1b, th2, m2 = _proj2(agg1, b1.reshape(1, 64), P2, npad)
    sa2b, sd2b, mvecs2 = _tables(th2, m2, valid, npad)
    al2 = _sc_logits(sa2b, sd2b, src1d, dst2d, mvecs2, zn, npad, epad)
    xwcat2 = jnp.concatenate([xw0b, xw1b], axis=0)
    agg2 = _sc_aggregate(xwcat2, al2, sidx2d, dst2d, zrows, npad, epad)

    out = _mlp(agg2, b2.reshape(1, 64), Wf1, bf1, Wf2, bf2, Wf3, bf3,
               Wf4, bf4, Wf5, bf5, npad)
    return out[:n]


# speedup vs baseline: 1.0852x; 1.0852x over previous
import functools

import jax
import jax.numpy as jnp
from jax import lax
from jax.experimental import pallas as pl
from jax.experimental.pallas import tpu as pltpu
from jax.experimental.pallas import tpu_sc as plsc

H = 2
C = 32
NC = 2
NS = 16
BLK = 1792
KA = 2048
KB = 256
NEG = -1e30


def _sc_mesh():
    return plsc.VectorSubcoreMesh(
        core_axis_name="c", subcore_axis_name="s", num_cores=NC, num_subcores=NS)


_SC_PARAMS = pltpu.CompilerParams(
    needs_layout_passes=False, use_tc_tiling_on_sc=False)


def _stats(hflat, nvalid):
    def body(x_ref, o_ref):
        x = x_ref[...]
        r = lax.broadcasted_iota(jnp.int32, x.shape, 0)
        l = lax.broadcasted_iota(jnp.int32, x.shape, 1)
        g = r * 128 + l
        valid = g < nvalid
        even = (g % 2) == 0
        x0 = jnp.where(valid & even, x, 0.0)
        x1 = jnp.where(valid & (~even), x, 0.0)
        s0, s1 = jnp.sum(x0), jnp.sum(x1)
        q0, q1 = jnp.sum(x0 * x0), jnp.sum(x1 * x1)
        lane = lax.broadcasted_iota(jnp.int32, (1, 128), 1)
        o_ref[...] = jnp.where(
            lane == 0, s0, jnp.where(lane == 1, s1, jnp.where(lane == 2, q0, q1)))
    return pl.pallas_call(
        body, out_shape=jax.ShapeDtypeStruct((1, 128), jnp.float32))(hflat)


def _proj1(hp4, Astk, rstk, Ath, rth, npad):
    nb = npad // BLK
    def body(x_ref, a_ref, r_ref, at_ref, rt_ref, xw_ref, th_ref, m_ref):
        x = x_ref[...]
        xw_ref[...] = jnp.dot(x, a_ref[0], preferred_element_type=jnp.float32,
                              precision=lax.Precision.HIGHEST) + r_ref[0]
        th = jnp.dot(x, at_ref[...], preferred_element_type=jnp.float32,
                     precision=lax.Precision.HIGHEST) + rt_ref[...]
        th_ref[...] = th
        cur = jnp.max(th, axis=0, keepdims=True)
        first = (pl.program_id(0) == 0) & (pl.program_id(1) == 0)
        @pl.when(first)
        def _():
            m_ref[...] = cur
        @pl.when(~first)
        def _():
            m_ref[...] = jnp.maximum(m_ref[...], cur)
    return pl.pallas_call(
        body, grid=(2, nb),
        in_specs=[
            pl.BlockSpec((BLK // 4, 8), lambda h, i: (i, 0)),
            pl.BlockSpec((1, 8, 128), lambda h, i: (h, 0, 0)),
            pl.BlockSpec((1, 1, 128), lambda h, i: (h, 0, 0)),
            pl.BlockSpec((8, 16), lambda h, i: (0, 0)),
            pl.BlockSpec((1, 16), lambda h, i: (0, 0)),
        ],
        out_specs=[
            pl.BlockSpec((BLK // 4, 128), lambda h, i: (h * nb + i, 0)),
            pl.BlockSpec((BLK // 4, 16), lambda h, i: (i, 0)),
            pl.BlockSpec((1, 16), lambda h, i: (0, 0)),
        ],
        out_shape=[
            jax.ShapeDtypeStruct((2 * npad // 4, 128), jnp.float32),
            jax.ShapeDtypeStruct((npad // 4, 16), jnp.float32),
            jax.ShapeDtypeStruct((1, 16), jnp.float32),
        ])(hp4, Astk, rstk, Ath, rth)


def _proj2(aggp, bpk, Astk, Ath, npad):
    nb = npad // BLK
    def body(a0_ref, a1_ref, b_ref, a_ref, at_ref, xw_ref, th_ref, m_ref):
        x = jnp.concatenate([a0_ref[...], a1_ref[...]], axis=1) + b_ref[...]
        x = jnp.maximum(x, 0.0)
        xw_ref[...] = jnp.dot(x, a_ref[0], preferred_element_type=jnp.float32,
                              precision=lax.Precision.HIGHEST)
        th = jnp.dot(x, at_ref[...], preferred_element_type=jnp.float32,
                     precision=lax.Precision.HIGHEST)
        th_ref[...] = th
        cur = jnp.max(th, axis=0, keepdims=True)
        first = (pl.program_id(0) == 0) & (pl.program_id(1) == 0)
        @pl.when(first)
        def _():
            m_ref[...] = cur
        @pl.when(~first)
        def _():
            m_ref[...] = jnp.maximum(m_ref[...], cur)
    return pl.pallas_call(
        body, grid=(2, nb),
        in_specs=[
            pl.BlockSpec((BLK // 4, 128), lambda h, i: (i, 0)),
            pl.BlockSpec((BLK // 4, 128), lambda h, i: (i + nb, 0)),
            pl.BlockSpec((1, 256), lambda h, i: (0, 0)),
            pl.BlockSpec((1, 256, 128), lambda h, i: (h, 0, 0)),
            pl.BlockSpec((256, 16), lambda h, i: (0, 0)),
        ],
        out_specs=[
            pl.BlockSpec((BLK // 4, 128), lambda h, i: (h * nb + i, 0)),
            pl.BlockSpec((BLK // 4, 16), lambda h, i: (i, 0)),
            pl.BlockSpec((1, 16), lambda h, i: (0, 0)),
        ],
        out_shape=[
            jax.ShapeDtypeStruct((2 * npad // 4, 128), jnp.float32),
            jax.ShapeDtypeStruct((npad // 4, 16), jnp.float32),
            jax.ShapeDtypeStruct((1, 16), jnp.float32),
        ])(aggp, aggp, bpk, Astk, Ath)


def _mlp(aggp, bpk, Wk1, ck1, Wk2, ck2, Wk3, ck3, Wk4, ck4, Wk5, ck5, npad):
    nb = npad // BLK
    def body(a0_ref, a1_ref, b_ref, w1, c1, w2, c2, w3, c3, w4, c4, w5, c5,
             out_ref):
        x = jnp.concatenate([a0_ref[...], a1_ref[...]], axis=1) + b_ref[...]
        x = jnp.maximum(x, 0.0)
        for w, c in ((w1, c1), (w2, c2), (w3, c3), (w4, c4)):
            x = jnp.maximum(
                jnp.dot(x, w[...], preferred_element_type=jnp.float32,
                        precision=lax.Precision.HIGHEST) + c[...],
                0.0)
        out_ref[...] = jnp.dot(x, w5[...], preferred_element_type=jnp.float32,
                               precision=lax.Precision.HIGHEST) + c5[...]
    full = lambda s: pl.BlockSpec(s, lambda i: (0, 0))
    return pl.pallas_call(
        body, grid=(nb,),
        in_specs=[
            pl.BlockSpec((BLK // 4, 128), lambda i: (i, 0)),
            pl.BlockSpec((BLK // 4, 128), lambda i: (i + nb, 0)),
            full((1, 256)),
            full((256, 256)), full((1, 256)),
            full((256, 256)), full((1, 256)),
            full((256, 256)), full((1, 256)),
            full((256, 256)), full((1, 256)),
            full((256, 64)), full((1, 64)),
        ],
        out_specs=pl.BlockSpec((BLK // 4, 64), lambda i: (i, 0)),
        out_shape=jax.ShapeDtypeStruct((npad // 4, 64), jnp.float32),
    )(aggp, aggp, bpk, Wk1, ck1, Wk2, ck2, Wk3, ck3, Wk4, ck4, Wk5, ck5)


def _sc_logits(sa2, sd2, src1d, dst2d, mvecs, zn, npad, epad):
    nslice = npad // NS
    ept = epad // NS
    nchunks = ept // KA

    @functools.partial(
        pl.kernel,
        out_type=jax.ShapeDtypeStruct((NC * epad,), jnp.float32),
        mesh=_sc_mesh(),
        compiler_params=_SC_PARAMS,
        scratch_types=[
            pltpu.VMEM((npad,), jnp.float32),
            pltpu.VMEM((npad,), jnp.float32),
            pltpu.VMEM((KA,), jnp.int32),
            pltpu.VMEM((KA,), jnp.int32),
            pltpu.VMEM((KA // 128, 128), jnp.int32),
            pltpu.VMEM((KA // 128, 128), jnp.int32),
            pltpu.VMEM((KA,), jnp.float32),
            pltpu.VMEM((KA,), jnp.float32),
            pltpu.VMEM((16,), jnp.float32),
            pltpu.VMEM((nslice,), jnp.float32),
            pltpu.VMEM_SHARED((npad,), jnp.float32),
            pltpu.SemaphoreType.DMA,
            pltpu.SemaphoreType.DMA,
        ])
    def kern(sa_h, sd_h, src_h, dst2_h, mv_h, zn_h, t_out,
             satab, sdtab, src0, src1, dst20, dst21, tb0, tb1, m_b,
             zslice, s_acc, ss0, ss1):
        c = lax.axis_index("c")
        s = lax.axis_index("s")
        pltpu.sync_copy(sa_h.at[pl.ds(c * npad, npad)], satab)
        pltpu.sync_copy(sd_h.at[pl.ds(c * npad, npad)], sdtab)
        pltpu.sync_copy(mv_h.at[pl.ds(c * 16, 16)], m_b)
        pltpu.sync_copy(zn_h, zslice)
        pltpu.sync_copy(zslice, s_acc.at[pl.ds(s * nslice, nslice)])
        plsc.subcore_barrier()
        mv = m_b[...]

        def load(k, src_b, dst2_b):
            off = pl.multiple_of(s * ept + k * KA, KA)
            row = s * (ept // 128) + k * (KA // 128)
            pltpu.sync_copy(src_h.at[pl.ds(off, KA)], src_b)
            pltpu.sync_copy(dst2_h.at[pl.ds(row, KA // 128), :], dst2_b)

        def compute(k, src_b, dst2_b, t_b, sem):
            off = pl.multiple_of(s * ept + k * KA, KA)

            def vec(v, carry2):
                o = pl.multiple_of(v * 16, 16)
                r = v // 8
                col = pl.multiple_of((v % 8) * 16, 16)
                a = plsc.load_gather(satab, [src_b[pl.ds(o, 16)]])
                d = plsc.load_gather(sdtab, [dst2_b[r, pl.ds(col, 16)]])
                e = a + d
                e = jnp.where(e > 0, e, 0.2 * e)
                t_b[pl.ds(o, 16)] = jnp.exp(e - mv)
                return carry2
            lax.fori_loop(0, KA // 16, vec, None)
            pltpu.sync_copy(t_b, t_out.at[pl.ds(c * epad + off, KA)])
            for b in range(KA // 128):
                pltpu.async_copy(t_b.at[pl.ds(b * 128, 128)],
                                 s_acc.at[dst2_b.at[b]], sem, add=True)

        def drain(t_b, sem):
            pltpu.make_async_copy(zn_h.at[pl.ds(0, KA)], t_b, sem).wait()

        load(0, src0, dst20)
        nch2 = nchunks // 2

        def body2(k2, carry):
            @pl.when(k2 > 0)
            def _():
                drain(tb1, ss1)
            load(2 * k2 + 1, src1, dst21)
            compute(2 * k2, src0, dst20, tb0, ss0)

            @pl.when(k2 + 1 < nch2)
            def _():
                drain(tb0, ss0)
                load(2 * k2 + 2, src0, dst20)
            compute(2 * k2 + 1, src1, dst21, tb1, ss1)
            return carry
        lax.fori_loop(0, nch2, body2, None)
        drain(tb0, ss0)
        drain(tb1, ss1)
        plsc.subcore_barrier()
        pltpu.sync_copy(s_acc, satab)

        def norm(k, dst2_b, t_b):
            off = pl.multiple_of(s * ept + k * KA, KA)
            pltpu.sync_copy(t_out.at[pl.ds(c * epad + off, KA)], t_b)

            def vec2(v, carry2):
                o = pl.multiple_of(v * 16, 16)
                r = v // 8
                col = pl.multiple_of((v % 8) * 16, 16)
                sv = plsc.load_gather(satab, [dst2_b[r, pl.ds(col, 16)]])
                t_b[pl.ds(o, 16)] = t_b[pl.ds(o, 16)] / (sv + 1e-16)
                return carry2
            lax.fori_loop(0, KA // 16, vec2, None)
            pltpu.sync_copy(t_b, t_out.at[pl.ds(c * epad + off, KA)])

        def load2(k, dst2_b):
            row = s * (ept // 128) + k * (KA // 128)
            pltpu.sync_copy(dst2_h.at[pl.ds(row, KA // 128), :], dst2_b)

        load2(0, dst20)

        def body2b(k2, carry):
            load2(2 * k2 + 1, dst21)
            norm(2 * k2, dst20, tb0)

            @pl.when(k2 + 1 < nch2)
            def _():
                load2(2 * k2 + 2, dst20)
            norm(2 * k2 + 1, dst21, tb1)
            return carry
        lax.fori_loop(0, nch2, body2b, None)

    return kern(sa2, sd2, src1d, dst2d, mvecs, zn)


def _sc_aggregate(xwcat, alpha, sidx2d, dst2d, zrows, npad, epad):
    nslice = npad // NS
    ept = epad // NS
    nchunks = ept // KB

    @functools.partial(
        pl.kernel,
        out_type=jax.ShapeDtypeStruct((NC * npad // 4, 128), jnp.float32),
        mesh=_sc_mesh(),
        compiler_params=_SC_PARAMS,
        scratch_types=[
            pltpu.VMEM((KB,), jnp.float32),
            pltpu.VMEM((KB,), jnp.float32),
            pltpu.VMEM((KB // 128, 128), jnp.int32),
            pltpu.VMEM((KB // 128, 128), jnp.int32),
            pltpu.VMEM((KB // 128, 128), jnp.int32),
            pltpu.VMEM((KB // 128, 128), jnp.int32),
            pltpu.VMEM((KB, 32), jnp.float32),
            pltpu.VMEM((KB, 32), jnp.float32),
            pltpu.VMEM((28, 128), jnp.float32),
            pltpu.VMEM_SHARED((npad, 32), jnp.float32),
            pltpu.SemaphoreType.DMA,
            pltpu.SemaphoreType.DMA,
            pltpu.SemaphoreType.DMA,
            pltpu.SemaphoreType.DMA,
        ])
    def kern(xw_h, al_h, sidx_h, dst2_h, z_h, agg_out,
             al0, al1, d20, d21, si0, si1, rows0, rows1, b128, acc,
             sg0, sg1, ss0, ss1):
        c = lax.axis_index("c")
        s = lax.axis_index("s")
        pltpu.sync_copy(z_h, rows0)
        for p in range(nslice // 224):
            pltpu.sync_copy(rows0.at[pl.ds(0, 224), :],
                            acc.at[pl.ds(s * nslice + p * 224, 224), :])
        plsc.subcore_barrier()

        def load(k, al_b, dst2_b, sidx_b, rows, sem):
            off = pl.multiple_of(s * ept + k * KB, KB)
            row = s * (ept // 128) + k * (KB // 128)
            pltpu.sync_copy(al_h.at[pl.ds(c * epad + off, KB)], al_b)
            pltpu.sync_copy(dst2_h.at[pl.ds(row, KB // 128), :], dst2_b)
            pltpu.sync_copy(
                sidx_h.at[pl.ds(c * (epad // 128) + row, KB // 128), :], sidx_b)
            for b in range(KB // 128):
                pltpu.async_copy(xw_h.at[sidx_b.at[b]],
                                 rows.at[pl.ds(b * 128, 128), :], sem)

        def drain(rows, sem):
            pltpu.make_async_copy(z_h, rows, sem).wait()

        def scale_scatter(al_b, dst2_b, rows, sem):
            def scale_g(g, carry2):
                o = pl.multiple_of(g * 16, 16)
                al = al_b[pl.ds(o, 16)]
                for j in range(16):
                    e = g * 16 + j
                    sp = al.at[jnp.full((16,), j, jnp.int32)].get(
                        mode="promise_in_bounds")
                    rows[e, pl.ds(0, 16)] = rows[e, pl.ds(0, 16)] * sp
                    rows[e, pl.ds(16, 16)] = rows[e, pl.ds(16, 16)] * sp
                return carry2
            lax.fori_loop(0, KB // 16, scale_g, None)
            for b in range(KB // 128):
                pltpu.async_copy(rows.at[pl.ds(b * 128, 128), :],
                                 acc.at[dst2_b.at[b]], sem, add=True)

        load(0, al0, d20, si0, rows0, sg0)
        nch2 = nchunks // 2

        def body2(k2, carry):
            @pl.when(k2 > 0)
            def _():
                drain(rows1, ss1)
            load(2 * k2 + 1, al1, d21, si1, rows1, sg1)
            drain(rows0, sg0)
            scale_scatter(al0, d20, rows0, ss0)

            @pl.when(k2 + 1 < nch2)
            def _():
                drain(rows0, ss0)
                load(2 * k2 + 2, al0, d20, si0, rows0, sg0)
            drain(rows1, sg1)
            scale_scatter(al1, d21, rows1, ss1)
            return carry
        lax.fori_loop(0, nch2, body2, None)
        drain(rows0, ss0)
        drain(rows1, ss1)
        plsc.subcore_barrier()
        base4 = (c * npad + s * nslice) // 4

        def wb(p, carry):
            pltpu.sync_copy(acc.at[pl.ds(s * nslice + p * 112, 112), :],
                            rows0.at[pl.ds(0, 112), :])
            for r in range(28):
                for q in range(8):
                    b128[r, pl.ds(q * 16, 16)] = (
                        rows0[4 * r + q // 2, pl.ds(16 * (q % 2), 16)])
            pltpu.sync_copy(b128, agg_out.at[pl.ds(base4 + p * 28, 28), :])
            return carry
        lax.fori_loop(0, nslice // 112, wb, None)

    return kern(xwcat, alpha, sidx2d, dst2d, zrows)


def _leaky(v):
    return jnp.where(v > 0, v, 0.2 * v)


def _attn_cols(W, a_s, a_d):
    cols = []
    for h in range(H):
        cols.append(W[:, h * C:(h + 1) * C] @ a_s[h])
        cols.append(W[:, h * C:(h + 1) * C] @ a_d[h])
    return jnp.stack(cols, axis=1)


def _kron4(W):
    return jnp.kron(jnp.eye(4, dtype=jnp.float32), W)


def _vk(Wtop, Wbot):
    return jnp.concatenate([_kron4(Wtop), _kron4(Wbot)], axis=0)


def _tile4(v):
    return jnp.tile(v, 4)


def _tables(th_p, m, valid, npad):
    th = th_p.reshape(npad, 4)
    sa2 = jnp.concatenate([jnp.where(valid, th[:, 0], NEG),
                           jnp.where(valid, th[:, 2], NEG)])
    sd2 = jnp.concatenate([jnp.where(valid, th[:, 1], 0.0),
                           jnp.where(valid, th[:, 3], 0.0)])
    mx = jnp.max(m[0].reshape(4, 4), axis=0)
    m0 = _leaky(mx[0] + mx[1])
    m1 = _leaky(mx[2] + mx[3])
    mvecs = jnp.concatenate([jnp.full((16,), m0, jnp.float32),
                             jnp.full((16,), m1, jnp.float32)])
    return sa2, sd2, mvecs


def kernel(h, edge_index, bn_g, bn_b, W1, as1, ad1, b1, W2, as2, ad2, b2,
           Wf1, bf1, Wf2, bf2, Wf3, bf3, Wf4, bf4, Wf5, bf5):
    n = h.shape[0]
    npad = ((n + 2 * BLK - 1) // (2 * BLK)) * (2 * BLK)
    e_tot = edge_index.shape[1] + n
    epad = ((e_tot + NS * KA - 1) // (NS * KA)) * (NS * KA)

    loops = jnp.arange(n, dtype=jnp.int32)
    pad_e = jnp.full((epad - e_tot,), n, jnp.int32)
    src1d = jnp.concatenate([edge_index[0].astype(jnp.int32), loops, pad_e])
    dst1d = jnp.concatenate([edge_index[1].astype(jnp.int32), loops, pad_e])
    dst2d = dst1d.reshape(epad // 128, 128)
    sidx2d = jnp.concatenate([src1d, src1d + npad]).reshape(2 * epad // 128, 128)
    zn = jnp.zeros((npad // NS,), jnp.float32)
    zrows = jnp.zeros((KB, 32), jnp.float32)
    valid = jnp.arange(npad) < n

    hpad = jnp.pad(h, ((0, npad - n), (0, 0)))
    stats = _stats(hpad.reshape(npad * 2 // 128, 128), 2 * n)
    mean = jnp.stack([stats[0, 0], stats[0, 1]]) / n
    msq = jnp.stack([stats[0, 2], stats[0, 3]]) / n
    var = msq - mean * mean
    scale = bn_g / jnp.sqrt(var + 1e-5)
    shift = bn_b - mean * scale
    W1e = W1 * scale[:, None]
    att1 = _attn_cols(W1e, as1, ad1)
    Astk1 = jnp.stack([_kron4(W1e[:, 32 * h:32 * h + 32]) for h in range(H)])
    r_xw = shift @ W1
    r_att = jnp.stack([r_xw[0:32] @ as1[0], r_xw[0:32] @ ad1[0],
                       r_xw[32:64] @ as1[1], r_xw[32:64] @ ad1[1]])
    rstk = jnp.stack([_tile4(r_xw[0:32]).reshape(1, 128),
                      _tile4(r_xw[32:64]).reshape(1, 128)])
    Ath1 = _kron4(att1)
    rth = _tile4(r_att).reshape(1, 16)

    hp4 = hpad.reshape(npad // 4, 8)
    xwp1, th1, m1 = _proj1(hp4, Astk1, rstk, Ath1, rth, npad)
    sa2, sd2, mvecs = _tables(th1, m1, valid, npad)
    al1 = _sc_logits(sa2, sd2, src1d, dst2d, mvecs, zn, npad, epad)
    agg1 = _sc_aggregate(xwp1.reshape(2 * npad, 32), al1, sidx2d, dst2d,
                         zrows, npad, epad)

    att2 = _attn_cols(W2, as2, ad2)
    Astk2 = jnp.stack([_vk(W2[0:32, 32 * h:32 * h + 32],
                           W2[32:64, 32 * h:32 * h + 32]) for h in range(H)])
    Ath2 = _vk(att2[0:32, :], att2[32:64, :])
    bpk1 = jnp.concatenate([_tile4(b1[0:32]), _tile4(b1[32:64])]).reshape(1, 256)
    xwp2, th2, m2 = _proj2(agg1, bpk1, Astk2, Ath2, npad)
    sa2b, sd2b, mvecs2 = _tables(th2, m2, valid, npad)
    al2 = _sc_logits(sa2b, sd2b, src1d, dst2d, mvecs2, zn, npad, epad)
    agg2 = _sc_aggregate(xwp2.reshape(2 * npad, 32), al2, sidx2d, dst2d,
                         zrows, npad, epad)

    bpk2 = jnp.concatenate([_tile4(b2[0:32]), _tile4(b2[32:64])]).reshape(1, 256)
    W5p = jnp.pad(Wf5, ((0, 0), (0, 5)))
    b5p = jnp.pad(bf5, (0, 5))
    outp = _mlp(
        agg2, bpk2,
        _vk(Wf1[0:32, :], Wf1[32:64, :]), _tile4(bf1).reshape(1, 256),
        _kron4(Wf2), _tile4(bf2).reshape(1, 256),
        _kron4(Wf3), _tile4(bf3).reshape(1, 256),
        _kron4(Wf4), _tile4(bf4).reshape(1, 256),
        _kron4(W5p), _tile4(b5p).reshape(1, 64),
        npad)
    return outp.reshape(npad, 16)[:n, :11]


# speedup vs baseline: 1.3188x; 1.2153x over previous
import functools

import jax
import jax.numpy as jnp
from jax import lax
from jax.experimental import pallas as pl
from jax.experimental.pallas import tpu as pltpu
from jax.experimental.pallas import tpu_sc as plsc

H = 2
C = 32
NC = 2
NS = 16
BLK = 1792
KA = 2048
KB = 128
NEG = -1e30


def _sc_mesh():
    return plsc.VectorSubcoreMesh(
        core_axis_name="c", subcore_axis_name="s", num_cores=NC, num_subcores=NS)


_SC_PARAMS = pltpu.CompilerParams(
    needs_layout_passes=False, use_tc_tiling_on_sc=False)


def _stats(hflat, nvalid):
    def body(x_ref, o_ref):
        x = x_ref[...]
        r = lax.broadcasted_iota(jnp.int32, x.shape, 0)
        l = lax.broadcasted_iota(jnp.int32, x.shape, 1)
        g = r * 128 + l
        valid = g < nvalid
        even = (g % 2) == 0
        x0 = jnp.where(valid & even, x, 0.0)
        x1 = jnp.where(valid & (~even), x, 0.0)
        s0, s1 = jnp.sum(x0), jnp.sum(x1)
        q0, q1 = jnp.sum(x0 * x0), jnp.sum(x1 * x1)
        lane = lax.broadcasted_iota(jnp.int32, (1, 128), 1)
        o_ref[...] = jnp.where(
            lane == 0, s0, jnp.where(lane == 1, s1, jnp.where(lane == 2, q0, q1)))
    return pl.pallas_call(
        body, out_shape=jax.ShapeDtypeStruct((1, 128), jnp.float32))(hflat)


def _proj1(hp4, Astk, rstk, Ath, rth, npad):
    nb = npad // BLK
    def body(x_ref, a_ref, r_ref, at_ref, rt_ref, xw_ref, th_ref, m_ref):
        x = x_ref[...]
        xw_ref[...] = jnp.dot(x, a_ref[0], preferred_element_type=jnp.float32,
                              precision=lax.Precision.HIGHEST) + r_ref[0]
        th = jnp.dot(x, at_ref[...], preferred_element_type=jnp.float32,
                     precision=lax.Precision.HIGHEST) + rt_ref[...]
        th_ref[...] = th
        cur = jnp.max(th, axis=0, keepdims=True)
        first = (pl.program_id(0) == 0) & (pl.program_id(1) == 0)
        @pl.when(first)
        def _():
            m_ref[...] = cur
        @pl.when(~first)
        def _():
            m_ref[...] = jnp.maximum(m_ref[...], cur)
    return pl.pallas_call(
        body, grid=(2, nb),
        in_specs=[
            pl.BlockSpec((BLK // 4, 8), lambda h, i: (i, 0)),
            pl.BlockSpec((1, 8, 128), lambda h, i: (h, 0, 0)),
            pl.BlockSpec((1, 1, 128), lambda h, i: (h, 0, 0)),
            pl.BlockSpec((8, 16), lambda h, i: (0, 0)),
            pl.BlockSpec((1, 16), lambda h, i: (0, 0)),
        ],
        out_specs=[
            pl.BlockSpec((BLK // 4, 128), lambda h, i: (h * nb + i, 0)),
            pl.BlockSpec((BLK // 4, 16), lambda h, i: (i, 0)),
            pl.BlockSpec((1, 16), lambda h, i: (0, 0)),
        ],
        out_shape=[
            jax.ShapeDtypeStruct((2 * npad // 4, 128), jnp.float32),
            jax.ShapeDtypeStruct((npad // 4, 16), jnp.float32),
            jax.ShapeDtypeStruct((1, 16), jnp.float32),
        ])(hp4, Astk, rstk, Ath, rth)


def _proj2(aggp, bpk, Astk, Ath, npad):
    nb = npad // BLK
    def body(a0_ref, a1_ref, b_ref, a_ref, at_ref, xw_ref, th_ref, m_ref):
        x = jnp.concatenate([a0_ref[...], a1_ref[...]], axis=1) + b_ref[...]
        x = jnp.maximum(x, 0.0)
        xw_ref[...] = jnp.dot(x, a_ref[0], preferred_element_type=jnp.float32,
                              precision=lax.Precision.HIGHEST)
        th = jnp.dot(x, at_ref[...], preferred_element_type=jnp.float32,
                     precision=lax.Precision.HIGHEST)
        th_ref[...] = th
        cur = jnp.max(th, axis=0, keepdims=True)
        first = (pl.program_id(0) == 0) & (pl.program_id(1) == 0)
        @pl.when(first)
        def _():
            m_ref[...] = cur
        @pl.when(~first)
        def _():
            m_ref[...] = jnp.maximum(m_ref[...], cur)
    return pl.pallas_call(
        body, grid=(2, nb),
        in_specs=[
            pl.BlockSpec((BLK // 4, 128), lambda h, i: (i, 0)),
            pl.BlockSpec((BLK // 4, 128), lambda h, i: (i + nb, 0)),
            pl.BlockSpec((1, 256), lambda h, i: (0, 0)),
            pl.BlockSpec((1, 256, 128), lambda h, i: (h, 0, 0)),
            pl.BlockSpec((256, 16), lambda h, i: (0, 0)),
        ],
        out_specs=[
            pl.BlockSpec((BLK // 4, 128), lambda h, i: (h * nb + i, 0)),
            pl.BlockSpec((BLK // 4, 16), lambda h, i: (i, 0)),
            pl.BlockSpec((1, 16), lambda h, i: (0, 0)),
        ],
        out_shape=[
            jax.ShapeDtypeStruct((2 * npad // 4, 128), jnp.float32),
            jax.ShapeDtypeStruct((npad // 4, 16), jnp.float32),
            jax.ShapeDtypeStruct((1, 16), jnp.float32),
        ])(aggp, aggp, bpk, Astk, Ath)


def _mlp(aggp, bpk, Wk1, ck1, Wk2, ck2, Wk3, ck3, Wk4, ck4, Wk5, ck5, npad):
    nb = npad // BLK
    def body(a0_ref, a1_ref, b_ref, w1, c1, w2, c2, w3, c3, w4, c4, w5, c5,
             out_ref):
        x = jnp.concatenate([a0_ref[...], a1_ref[...]], axis=1) + b_ref[...]
        x = jnp.maximum(x, 0.0)
        for w, c in ((w1, c1), (w2, c2), (w3, c3), (w4, c4)):
            x = jnp.maximum(
                jnp.dot(x, w[...], preferred_element_type=jnp.float32,
                        precision=lax.Precision.HIGHEST) + c[...],
                0.0)
        out_ref[...] = jnp.dot(x, w5[...], preferred_element_type=jnp.float32,
                               precision=lax.Precision.HIGHEST) + c5[...]
    full = lambda s: pl.BlockSpec(s, lambda i: (0, 0))
    return pl.pallas_call(
        body, grid=(nb,),
        in_specs=[
            pl.BlockSpec((BLK // 4, 128), lambda i: (i, 0)),
            pl.BlockSpec((BLK // 4, 128), lambda i: (i + nb, 0)),
            full((1, 256)),
            full((256, 256)), full((1, 256)),
            full((256, 256)), full((1, 256)),
            full((256, 256)), full((1, 256)),
            full((256, 256)), full((1, 256)),
            full((256, 64)), full((1, 64)),
        ],
        out_specs=pl.BlockSpec((BLK // 4, 64), lambda i: (i, 0)),
        out_shape=jax.ShapeDtypeStruct((npad // 4, 64), jnp.float32),
    )(aggp, aggp, bpk, Wk1, ck1, Wk2, ck2, Wk3, ck3, Wk4, ck4, Wk5, ck5)


def _sc_logits(sa2, sd2, src1d, dst2d, mvecs, zn, npad, epad):
    nslice = npad // NS
    ept = epad // NS
    nchunks = ept // KA

    @functools.partial(
        pl.kernel,
        out_type=jax.ShapeDtypeStruct((NC * epad,), jnp.float32),
        mesh=_sc_mesh(),
        compiler_params=_SC_PARAMS,
        scratch_types=[
            pltpu.VMEM((npad,), jnp.float32),
            pltpu.VMEM((npad,), jnp.float32),
            pltpu.VMEM((KA,), jnp.int32),
            pltpu.VMEM((KA,), jnp.int32),
            pltpu.VMEM((KA // 128, 128), jnp.int32),
            pltpu.VMEM((KA // 128, 128), jnp.int32),
            pltpu.VMEM((KA,), jnp.float32),
            pltpu.VMEM((KA,), jnp.float32),
            pltpu.VMEM((16,), jnp.float32),
            pltpu.VMEM((nslice,), jnp.float32),
            pltpu.VMEM_SHARED((npad,), jnp.float32),
            pltpu.SemaphoreType.DMA,
            pltpu.SemaphoreType.DMA,
        ])
    def kern(sa_h, sd_h, src_h, dst2_h, mv_h, zn_h, t_out,
             satab, sdtab, src0, src1, dst20, dst21, tb0, tb1, m_b,
             zslice, s_acc, ss0, ss1):
        c = lax.axis_index("c")
        s = lax.axis_index("s")
        pltpu.sync_copy(sa_h.at[pl.ds(c * npad, npad)], satab)
        pltpu.sync_copy(sd_h.at[pl.ds(c * npad, npad)], sdtab)
        pltpu.sync_copy(mv_h.at[pl.ds(c * 16, 16)], m_b)
        pltpu.sync_copy(zn_h, zslice)
        pltpu.sync_copy(zslice, s_acc.at[pl.ds(s * nslice, nslice)])
        plsc.subcore_barrier()
        mv = m_b[...]

        def load(k, src_b, dst2_b):
            off = pl.multiple_of(s * ept + k * KA, KA)
            row = s * (ept // 128) + k * (KA // 128)
            pltpu.sync_copy(src_h.at[pl.ds(off, KA)], src_b)
            pltpu.sync_copy(dst2_h.at[pl.ds(row, KA // 128), :], dst2_b)

        def compute(k, src_b, dst2_b, t_b, sem):
            off = pl.multiple_of(s * ept + k * KA, KA)

            def vec(v, carry2):
                o = pl.multiple_of(v * 16, 16)
                r = v // 8
                col = pl.multiple_of((v % 8) * 16, 16)
                a = plsc.load_gather(satab, [src_b[pl.ds(o, 16)]])
                d = plsc.load_gather(sdtab, [dst2_b[r, pl.ds(col, 16)]])
                e = a + d
                e = jnp.where(e > 0, e, 0.2 * e)
                t_b[pl.ds(o, 16)] = jnp.exp(e - mv)
                return carry2
            lax.fori_loop(0, KA // 16, vec, None)
            pltpu.sync_copy(t_b, t_out.at[pl.ds(c * epad + off, KA)])
            for b in range(KA // 128):
                pltpu.async_copy(t_b.at[pl.ds(b * 128, 128)],
                                 s_acc.at[dst2_b.at[b]], sem, add=True)

        def drain(t_b, sem):
            pltpu.make_async_copy(zn_h.at[pl.ds(0, KA)], t_b, sem).wait()

        load(0, src0, dst20)
        nch2 = nchunks // 2

        def body2(k2, carry):
            @pl.when(k2 > 0)
            def _():
                drain(tb1, ss1)
            load(2 * k2 + 1, src1, dst21)
            compute(2 * k2, src0, dst20, tb0, ss0)

            @pl.when(k2 + 1 < nch2)
            def _():
                drain(tb0, ss0)
                load(2 * k2 + 2, src0, dst20)
            compute(2 * k2 + 1, src1, dst21, tb1, ss1)
            return carry
        lax.fori_loop(0, nch2, body2, None)
        drain(tb0, ss0)
        drain(tb1, ss1)
        plsc.subcore_barrier()
        pltpu.sync_copy(s_acc, satab)

        def norm(k, dst2_b, t_b):
            off = pl.multiple_of(s * ept + k * KA, KA)
            pltpu.sync_copy(t_out.at[pl.ds(c * epad + off, KA)], t_b)

            def vec2(v, carry2):
                o = pl.multiple_of(v * 16, 16)
                r = v // 8
                col = pl.multiple_of((v % 8) * 16, 16)
                sv = plsc.load_gather(satab, [dst2_b[r, pl.ds(col, 16)]])
                t_b[pl.ds(o, 16)] = t_b[pl.ds(o, 16)] / (sv + 1e-16)
                return carry2
            lax.fori_loop(0, KA // 16, vec2, None)
            pltpu.sync_copy(t_b, t_out.at[pl.ds(c * epad + off, KA)])

        def load2(k, dst2_b):
            row = s * (ept // 128) + k * (KA // 128)
            pltpu.sync_copy(dst2_h.at[pl.ds(row, KA // 128), :], dst2_b)

        load2(0, dst20)

        def body2b(k2, carry):
            load2(2 * k2 + 1, dst21)
            norm(2 * k2, dst20, tb0)

            @pl.when(k2 + 1 < nch2)
            def _():
                load2(2 * k2 + 2, dst20)
            norm(2 * k2 + 1, dst21, tb1)
            return carry
        lax.fori_loop(0, nch2, body2b, None)

    return kern(sa2, sd2, src1d, dst2d, mvecs, zn)


def _sc_aggregate(xwcat, alpha, sidx2d, dst2d, zrows, npad, epad):
    nslice = npad // NS
    ept = epad // NS
    nchunks = ept // KB

    @functools.partial(
        pl.kernel,
        out_type=jax.ShapeDtypeStruct((NC * npad // 4, 128), jnp.float32),
        mesh=_sc_mesh(),
        compiler_params=_SC_PARAMS,
        scratch_types=[
            pltpu.VMEM((16 * KB,), jnp.float32),
            pltpu.VMEM((16, 128), jnp.int32),
            pltpu.VMEM((16, 128), jnp.int32),
            pltpu.VMEM((KB, 32), jnp.float32),
            pltpu.VMEM((KB, 32), jnp.float32),
            pltpu.VMEM((KB, 32), jnp.float32),
            pltpu.VMEM((28, 128), jnp.float32),
            pltpu.VMEM_SHARED((npad, 32), jnp.float32),
            pltpu.SemaphoreType.DMA,
            pltpu.SemaphoreType.DMA,
            pltpu.SemaphoreType.DMA,
            pltpu.SemaphoreType.DMA,
            pltpu.SemaphoreType.DMA,
            pltpu.SemaphoreType.DMA,
        ])
    def kern(xw_h, al_h, sidx_h, dst2_h, z_h, agg_out,
             al_blk, d2_blk, si_blk, rows0, rows1, rows2, b128, acc,
             sg0, sg1, sg2, ss0, ss1, ss2):
        c = lax.axis_index("c")
        s = lax.axis_index("s")
        ROWS = (rows0, rows1, rows2)
        SG = (sg0, sg1, sg2)
        SS = (ss0, ss1, ss2)
        pltpu.sync_copy(z_h, rows0)
        for p in range(nslice // 112):
            pltpu.sync_copy(rows0.at[pl.ds(0, 112), :],
                            acc.at[pl.ds(s * nslice + p * 112, 112), :])
        plsc.subcore_barrier()

        def load_group(g):
            off = pl.multiple_of(s * ept + g * 16 * KB, KB)
            row = s * (ept // 128) + g * 16
            pltpu.sync_copy(al_h.at[pl.ds(c * epad + off, 16 * KB)], al_blk)
            pltpu.sync_copy(dst2_h.at[pl.ds(row, 16), :], d2_blk)
            pltpu.sync_copy(
                sidx_h.at[pl.ds(c * (epad // 128) + row, 16), :], si_blk)

        def fire_gather(j):
            pltpu.async_copy(xw_h.at[si_blk.at[j]],
                             ROWS[j % 3], SG[j % 3])

        def drain(sem, rows):
            pltpu.make_async_copy(z_h, rows, sem).wait()

        def scale_scatter(j):
            rows = ROWS[j % 3]

            def scale_g(g2, carry2):
                o = pl.multiple_of(j * KB + g2 * 16, 16)
                al = al_blk[pl.ds(o, 16)]
                for jj in range(16):
                    e = g2 * 16 + jj
                    sp = al.at[jnp.full((16,), jj, jnp.int32)].get(
                        mode="promise_in_bounds")
                    rows[e, pl.ds(0, 16)] = rows[e, pl.ds(0, 16)] * sp
                    rows[e, pl.ds(16, 16)] = rows[e, pl.ds(16, 16)] * sp
                return carry2
            lax.fori_loop(0, KB // 16, scale_g, None)
            pltpu.async_copy(rows, acc.at[d2_blk.at[j]], SS[j % 3], add=True)

        ngroups = nchunks // 16

        def group_body(g, carry):
            load_group(g)
            fire_gather(0)
            fire_gather(1)
            for j in range(16):
                m = j % 3
                drain(SG[m], ROWS[m])
                scale_scatter(j)
                if j + 2 < 16:
                    m2 = (j + 2) % 3
                    if j > 0:
                        drain(SS[m2], ROWS[m2])
                    fire_gather(j + 2)
            for j in (13, 14, 15):
                drain(SS[j % 3], ROWS[j % 3])
            return carry
        lax.fori_loop(0, ngroups, group_body, None)
        plsc.subcore_barrier()
        base4 = (c * npad + s * nslice) // 4

        def wb(p, carry):
            pltpu.sync_copy(acc.at[pl.ds(s * nslice + p * 112, 112), :],
                            rows0.at[pl.ds(0, 112), :])
            for r in range(28):
                for q in range(8):
                    b128[r, pl.ds(q * 16, 16)] = (
                        rows0[4 * r + q // 2, pl.ds(16 * (q % 2), 16)])
            pltpu.sync_copy(b128, agg_out.at[pl.ds(base4 + p * 28, 28), :])
            return carry
        lax.fori_loop(0, nslice // 112, wb, None)

    return kern(xwcat, alpha, sidx2d, dst2d, zrows)


def _leaky(v):
    return jnp.where(v > 0, v, 0.2 * v)


def _attn_cols(W, a_s, a_d):
    cols = []
    for h in range(H):
        cols.append(W[:, h * C:(h + 1) * C] @ a_s[h])
        cols.append(W[:, h * C:(h + 1) * C] @ a_d[h])
    return jnp.stack(cols, axis=1)


def _kron4(W):
    return jnp.kron(jnp.eye(4, dtype=jnp.float32), W)


def _vk(Wtop, Wbot):
    return jnp.concatenate([_kron4(Wtop), _kron4(Wbot)], axis=0)


def _tile4(v):
    return jnp.tile(v, 4)


def _tables(th_p, m, valid, npad):
    th = th_p.reshape(npad, 4)
    sa2 = jnp.concatenate([jnp.where(valid, th[:, 0], NEG),
                           jnp.where(valid, th[:, 2], NEG)])
    sd2 = jnp.concatenate([jnp.where(valid, th[:, 1], 0.0),
                           jnp.where(valid, th[:, 3], 0.0)])
    mx = jnp.max(m[0].reshape(4, 4), axis=0)
    m0 = _leaky(mx[0] + mx[1])
    m1 = _leaky(mx[2] + mx[3])
    mvecs = jnp.concatenate([jnp.full((16,), m0, jnp.float32),
                             jnp.full((16,), m1, jnp.float32)])
    return sa2, sd2, mvecs


def kernel(h, edge_index, bn_g, bn_b, W1, as1, ad1, b1, W2, as2, ad2, b2,
           Wf1, bf1, Wf2, bf2, Wf3, bf3, Wf4, bf4, Wf5, bf5):
    n = h.shape[0]
    npad = ((n + 2 * BLK - 1) // (2 * BLK)) * (2 * BLK)
    e_tot = edge_index.shape[1] + n
    epad = ((e_tot + NS * KA - 1) // (NS * KA)) * (NS * KA)

    loops = jnp.arange(n, dtype=jnp.int32)
    pad_e = jnp.full((epad - e_tot,), n, jnp.int32)
    src1d = jnp.concatenate([edge_index[0].astype(jnp.int32), loops, pad_e])
    dst1d = jnp.concatenate([edge_index[1].astype(jnp.int32), loops, pad_e])
    dst2d = dst1d.reshape(epad // 128, 128)
    sidx2d = jnp.concatenate([src1d, src1d + npad]).reshape(2 * epad // 128, 128)
    zn = jnp.zeros((npad // NS,), jnp.float32)
    zrows = jnp.zeros((KB, 32), jnp.float32)
    valid = jnp.arange(npad) < n

    hpad = jnp.pad(h, ((0, npad - n), (0, 0)))
    stats = _stats(hpad.reshape(npad * 2 // 128, 128), 2 * n)
    mean = jnp.stack([stats[0, 0], stats[0, 1]]) / n
    msq = jnp.stack([stats[0, 2], stats[0, 3]]) / n
    var = msq - mean * mean
    scale = bn_g / jnp.sqrt(var + 1e-5)
    shift = bn_b - mean * scale
    W1e = W1 * scale[:, None]
    att1 = _attn_cols(W1e, as1, ad1)
    Astk1 = jnp.stack([_kron4(W1e[:, 32 * h:32 * h + 32]) for h in range(H)])
    r_xw = shift @ W1
    r_att = jnp.stack([r_xw[0:32] @ as1[0], r_xw[0:32] @ ad1[0],
                       r_xw[32:64] @ as1[1], r_xw[32:64] @ ad1[1]])
    rstk = jnp.stack([_tile4(r_xw[0:32]).reshape(1, 128),
                      _tile4(r_xw[32:64]).reshape(1, 128)])
    Ath1 = _kron4(att1)
    rth = _tile4(r_att).reshape(1, 16)

    hp4 = hpad.reshape(npad // 4, 8)
    xwp1, th1, m1 = _proj1(hp4, Astk1, rstk, Ath1, rth, npad)
    sa2, sd2, mvecs = _tables(th1, m1, valid, npad)
    al1 = _sc_logits(sa2, sd2, src1d, dst2d, mvecs, zn, npad, epad)
    agg1 = _sc_aggregate(xwp1.reshape(2 * npad, 32), al1, sidx2d, dst2d,
                         zrows, npad, epad)

    att2 = _attn_cols(W2, as2, ad2)
    Astk2 = jnp.stack([_vk(W2[0:32, 32 * h:32 * h + 32],
                           W2[32:64, 32 * h:32 * h + 32]) for h in range(H)])
    Ath2 = _vk(att2[0:32, :], att2[32:64, :])
    bpk1 = jnp.concatenate([_tile4(b1[0:32]), _tile4(b1[32:64])]).reshape(1, 256)
    xwp2, th2, m2 = _proj2(agg1, bpk1, Astk2, Ath2, npad)
    sa2b, sd2b, mvecs2 = _tables(th2, m2, valid, npad)
    al2 = _sc_logits(sa2b, sd2b, src1d, dst2d, mvecs2, zn, npad, epad)
    agg2 = _sc_aggregate(xwp2.reshape(2 * npad, 32), al2, sidx2d, dst2d,
                         zrows, npad, epad)

    bpk2 = jnp.concatenate([_tile4(b2[0:32]), _tile4(b2[32:64])]).reshape(1, 256)
    W5p = jnp.pad(Wf5, ((0, 0), (0, 5)))
    b5p = jnp.pad(bf5, (0, 5))
    outp = _mlp(
        agg2, bpk2,
        _vk(Wf1[0:32, :], Wf1[32:64, :]), _tile4(bf1).reshape(1, 256),
        _kron4(Wf2), _tile4(bf2).reshape(1, 256),
        _kron4(Wf3), _tile4(bf3).reshape(1, 256),
        _kron4(Wf4), _tile4(bf4).reshape(1, 256),
        _kron4(W5p), _tile4(b5p).reshape(1, 64),
        npad)
    return outp.reshape(npad, 16)[:n, :11]
